# Initial kernel scaffold; baseline (speedup 1.0000x reference)
import functools

import jax
import jax.numpy as jnp
from jax import lax
from jax.experimental import pallas as pl
from jax.experimental.pallas import tpu as pltpu
from jax.experimental.pallas import tpu_sc as plsc

N_B2 = 10000
N_B1 = 10000
E = 320000
D = 128
H = 128

NC = 2
NS = 16
LANES = 16
NW = NC * NS
CHUNK = 128
CHUNKS_PER_W = 80
E_PAD = NW * CHUNKS_PER_W * CHUNK
N_PAD = 10240
ROWS_PER_TILE = N_PAD // NS
ROW_BLK = 512
N_BLOCKS = N_PAD // ROW_BLK


def _sc_segment_sum(x_b2, src_w, dst_w):
  mesh = plsc.VectorSubcoreMesh(core_axis_name="c", subcore_axis_name="s")

  @functools.partial(
      pl.kernel,
      out_type=(
          jax.ShapeDtypeStruct((NC, N_PAD, D), jnp.float32),
          jax.ShapeDtypeStruct((NW, N_PAD), jnp.float32),
      ),
      mesh=mesh,
      scratch_types=[
          pltpu.VMEM((CHUNKS_PER_W, CHUNK), jnp.int32),
          pltpu.VMEM((CHUNKS_PER_W, CHUNK), jnp.int32),
          pltpu.VMEM((CHUNK, D), jnp.float32),
          pltpu.VMEM((N_PAD,), jnp.float32),
          pltpu.VMEM_SHARED((N_PAD, D), jnp.float32),
          pltpu.SemaphoreType.DMA,
      ],
  )
  def k(x_hbm, src_hbm, dst_hbm, psum_hbm, pcnt_hbm,
        src_v, dst_v, rows_v, cnt_v, acc_sh, sem):
    c = lax.axis_index("c")
    s = lax.axis_index("s")
    wid = s * NC + c

    zero16 = jnp.zeros((LANES,), jnp.float32)
    one16 = jnp.ones((LANES,), jnp.float32)

    def zb(t, _):
      rows_v[t // 8, pl.ds((t % 8) * LANES, LANES)] = zero16
      return 0
    lax.fori_loop(0, CHUNK * 8, zb, 0)

    def zc(t, _):
      cnt_v[pl.ds(t * LANES, LANES)] = zero16
      return 0
    lax.fori_loop(0, N_PAD // LANES, zc, 0)

    for kk in range(ROWS_PER_TILE // CHUNK):
      pltpu.sync_copy(rows_v, acc_sh.at[pl.ds(s * ROWS_PER_TILE + kk * CHUNK, CHUNK)])
    plsc.subcore_barrier()

    pltpu.sync_copy(src_hbm.at[wid], src_v)
    pltpu.sync_copy(dst_hbm.at[wid], dst_v)

    def chunk_body(j, _):
      cp = pltpu.async_copy(x_hbm.at[src_v.at[j]], rows_v, sem)
      for i in range(CHUNK // LANES):
        idx = dst_v[j, pl.ds(i * LANES, LANES)]
        plsc.addupdate_scatter(cnt_v, [idx], one16)
      cp.wait()
      pltpu.sync_copy(rows_v, acc_sh.at[dst_v.at[j]], add=True)
      return 0
    lax.fori_loop(0, CHUNKS_PER_W, chunk_body, 0)

    plsc.subcore_barrier()

    pltpu.sync_copy(acc_sh.at[pl.ds(s * ROWS_PER_TILE, ROWS_PER_TILE)],
                    psum_hbm.at[c, pl.ds(s * ROWS_PER_TILE, ROWS_PER_TILE)])
    pltpu.sync_copy(cnt_v, pcnt_hbm.at[wid])

  return k(x_b2, src_w, dst_w)


def _tc_body(psum_ref, pcnt_ref, x_ref, wl_ref, wr_ref, bl_ref,
             w1_ref, b1_ref, w2_ref, b2_ref, w3_ref, b3_ref, out_ref):
  summed = psum_ref[0] + psum_ref[1]
  cnt = jnp.sum(pcnt_ref[...], axis=0)
  mean = summed / jnp.maximum(cnt, 1.0)[:, None]
  h = (jnp.dot(mean, wl_ref[...], preferred_element_type=jnp.float32)
       + jnp.dot(x_ref[...], wr_ref[...], preferred_element_type=jnp.float32)
       + bl_ref[...])
  h = jnp.dot(h, w1_ref[...], preferred_element_type=jnp.float32) + b1_ref[...]
  h = jnp.where(h > 0, h, 0.01 * h)
  h = jnp.dot(h, w2_ref[...], preferred_element_type=jnp.float32) + b2_ref[...]
  h = jnp.where(h > 0, h, 0.01 * h)
  out_ref[...] = jnp.dot(h, w3_ref[...], preferred_element_type=jnp.float32) + b3_ref[...]


def kernel(x_B2, x_B1, edge_index, W_l, b_l, W_r, W1, b1, g1, be1, W2, b2, g2, be2, W3, b3):
  src = jnp.concatenate([edge_index[0].astype(jnp.int32),
                         jnp.zeros((E_PAD - E,), jnp.int32)])
  dst = jnp.concatenate([edge_index[1].astype(jnp.int32),
                         jnp.full((E_PAD - E,), N_B1, jnp.int32)])
  src_w = src.reshape(NW, CHUNKS_PER_W, CHUNK)
  dst_w = dst.reshape(NW, CHUNKS_PER_W, CHUNK)

  psum, pcnt = _sc_segment_sum(x_B2, src_w, dst_w)

  x_pad = jnp.concatenate(
      [x_B1, jnp.zeros((N_PAD - N_B1, D), jnp.float32)], axis=0)

  eps = 1e-5
  s1 = g1 / jnp.sqrt(1.0 + eps)
  s2 = g2 / jnp.sqrt(1.0 + eps)
  w1f = (W1 * s1[:, None]).T
  b1f = b1 * s1 + be1
  w2f = (W2 * s2[:, None]).T
  b2f = b2 * s2 + be2
  w3t = W3.T
  wlt = W_l.T
  wrt = W_r.T

  out = pl.pallas_call(
      _tc_body,
      grid=(N_BLOCKS,),
      in_specs=[
          pl.BlockSpec((NC, ROW_BLK, D), lambda i: (0, i, 0)),
          pl.BlockSpec((NW, ROW_BLK), lambda i: (0, i)),
          pl.BlockSpec((ROW_BLK, D), lambda i: (i, 0)),
          pl.BlockSpec((D, H), lambda i: (0, 0)),
          pl.BlockSpec((D, H), lambda i: (0, 0)),
          pl.BlockSpec((H,), lambda i: (0,)),
          pl.BlockSpec((H, 1280), lambda i: (0, 0)),
          pl.BlockSpec((1280,), lambda i: (0,)),
          pl.BlockSpec((1280, 480), lambda i: (0, 0)),
          pl.BlockSpec((480,), lambda i: (0,)),
          pl.BlockSpec((480, 1), lambda i: (0, 0)),
          pl.BlockSpec((1,), lambda i: (0,)),
      ],
      out_specs=pl.BlockSpec((ROW_BLK, 1), lambda i: (i, 0)),
      out_shape=jax.ShapeDtypeStruct((N_PAD, 1), jnp.float32),
  )(psum, pcnt, x_pad, wlt, wrt, b_l, w1f, b1f, w2f, b2f, w3t, b3)

  return out[:N_B1, 0]


# speedup vs baseline: 3.8426x; 3.8426x over previous
import functools

import jax
import jax.numpy as jnp
from jax import lax
from jax.experimental import pallas as pl
from jax.experimental.pallas import tpu as pltpu
from jax.experimental.pallas import tpu_sc as plsc

N_B2 = 10000
N_B1 = 10000
E = 320000
D = 128
H = 128

NC = 2
NS = 16
LANES = 16
NW = NC * NS
CHUNK = 128
CHUNKS_PER_W = 80
E_PAD = NW * CHUNKS_PER_W * CHUNK
N_PAD = 10240
ROWS_PER_TILE = N_PAD // NS
ROW_BLK = 512
N_BLOCKS = N_PAD // ROW_BLK


def _sc_segment_sum(x_b2, src_w, dst_w):
  mesh = plsc.VectorSubcoreMesh(core_axis_name="c", subcore_axis_name="s")

  @functools.partial(
      pl.kernel,
      out_type=(
          jax.ShapeDtypeStruct((NC, N_PAD, D), jnp.float32),
          jax.ShapeDtypeStruct((NW, N_PAD), jnp.float32),
      ),
      mesh=mesh,
      compiler_params=pltpu.CompilerParams(needs_layout_passes=False),
      scratch_types=[
          pltpu.VMEM((CHUNKS_PER_W, CHUNK), jnp.int32),
          pltpu.VMEM((CHUNKS_PER_W, CHUNK), jnp.int32),
          pltpu.VMEM((CHUNK, D), jnp.float32),
          pltpu.VMEM((N_PAD,), jnp.float32),
          pltpu.VMEM_SHARED((N_PAD, D), jnp.float32),
          pltpu.SemaphoreType.DMA,
      ],
  )
  def k(x_hbm, src_hbm, dst_hbm, psum_hbm, pcnt_hbm,
        src_v, dst_v, rows_v, cnt_v, acc_sh, sem):
    c = lax.axis_index("c")
    s = lax.axis_index("s")
    wid = s * NC + c

    zero16 = jnp.zeros((LANES,), jnp.float32)
    one16 = jnp.ones((LANES,), jnp.float32)

    def zb(t, _):
      rows_v[t // 8, pl.ds((t % 8) * LANES, LANES)] = zero16
      return 0
    lax.fori_loop(0, CHUNK * 8, zb, 0)

    def zc(t, _):
      cnt_v[pl.ds(t * LANES, LANES)] = zero16
      return 0
    lax.fori_loop(0, N_PAD // LANES, zc, 0)

    for kk in range(ROWS_PER_TILE // CHUNK):
      pltpu.sync_copy(rows_v, acc_sh.at[pl.ds(s * ROWS_PER_TILE + kk * CHUNK, CHUNK)])
    plsc.subcore_barrier()

    pltpu.sync_copy(src_hbm.at[wid], src_v)
    pltpu.sync_copy(dst_hbm.at[wid], dst_v)

    def chunk_body(j, _):
      cp = pltpu.async_copy(x_hbm.at[src_v.at[j]], rows_v, sem)
      for i in range(CHUNK // LANES):
        idx = dst_v[j, pl.ds(i * LANES, LANES)]
        plsc.addupdate_scatter(cnt_v, [idx], one16)
      cp.wait()
      pltpu.sync_copy(rows_v, acc_sh.at[dst_v.at[j]], add=True)
      return 0
    lax.fori_loop(0, CHUNKS_PER_W, chunk_body, 0)

    plsc.subcore_barrier()

    pltpu.sync_copy(acc_sh.at[pl.ds(s * ROWS_PER_TILE, ROWS_PER_TILE)],
                    psum_hbm.at[c, pl.ds(s * ROWS_PER_TILE, ROWS_PER_TILE)])
    pltpu.sync_copy(cnt_v, pcnt_hbm.at[wid])

  return k(x_b2, src_w, dst_w)


def _tc_body(psum_ref, pcnt_ref, x_ref, wl_ref, wr_ref, bl_ref,
             w1_ref, b1_ref, w2_ref, b2_ref, w3_ref, b3_ref, out_ref):
  summed = psum_ref[0] + psum_ref[1]
  cnt = jnp.sum(pcnt_ref[...], axis=0)
  mean = summed / jnp.maximum(cnt, 1.0)[:, None]
  h = (jnp.dot(mean, wl_ref[...], preferred_element_type=jnp.float32)
       + jnp.dot(x_ref[...], wr_ref[...], preferred_element_type=jnp.float32)
       + bl_ref[...])
  h = jnp.dot(h, w1_ref[...], preferred_element_type=jnp.float32) + b1_ref[...]
  h = jnp.where(h > 0, h, 0.01 * h)
  h = jnp.dot(h, w2_ref[...], preferred_element_type=jnp.float32) + b2_ref[...]
  h = jnp.where(h > 0, h, 0.01 * h)
  out_ref[...] = jnp.dot(h, w3_ref[...], preferred_element_type=jnp.float32) + b3_ref[...]


def kernel(x_B2, x_B1, edge_index, W_l, b_l, W_r, W1, b1, g1, be1, W2, b2, g2, be2, W3, b3):
  src = jnp.concatenate([edge_index[0].astype(jnp.int32),
                         jnp.zeros((E_PAD - E,), jnp.int32)])
  dst = jnp.concatenate([edge_index[1].astype(jnp.int32),
                         jnp.full((E_PAD - E,), N_B1, jnp.int32)])
  src_w = src.reshape(NW, CHUNKS_PER_W, CHUNK)
  dst_w = dst.reshape(NW, CHUNKS_PER_W, CHUNK)

  psum, pcnt = _sc_segment_sum(x_B2, src_w, dst_w)

  x_pad = jnp.concatenate(
      [x_B1, jnp.zeros((N_PAD - N_B1, D), jnp.float32)], axis=0)

  eps = 1e-5
  s1 = g1 / jnp.sqrt(1.0 + eps)
  s2 = g2 / jnp.sqrt(1.0 + eps)
  w1f = (W1 * s1[:, None]).T
  b1f = b1 * s1 + be1
  w2f = (W2 * s2[:, None]).T
  b2f = b2 * s2 + be2
  w3t = W3.T
  wlt = W_l.T
  wrt = W_r.T

  out = pl.pallas_call(
      _tc_body,
      grid=(N_BLOCKS,),
      in_specs=[
          pl.BlockSpec((NC, ROW_BLK, D), lambda i: (0, i, 0)),
          pl.BlockSpec((NW, ROW_BLK), lambda i: (0, i)),
          pl.BlockSpec((ROW_BLK, D), lambda i: (i, 0)),
          pl.BlockSpec((D, H), lambda i: (0, 0)),
          pl.BlockSpec((D, H), lambda i: (0, 0)),
          pl.BlockSpec((H,), lambda i: (0,)),
          pl.BlockSpec((H, 1280), lambda i: (0, 0)),
          pl.BlockSpec((1280,), lambda i: (0,)),
          pl.BlockSpec((1280, 480), lambda i: (0, 0)),
          pl.BlockSpec((480,), lambda i: (0,)),
          pl.BlockSpec((480, 1), lambda i: (0, 0)),
          pl.BlockSpec((1,), lambda i: (0,)),
      ],
      out_specs=pl.BlockSpec((ROW_BLK, 1), lambda i: (i, 0)),
      out_shape=jax.ShapeDtypeStruct((N_PAD, 1), jnp.float32),
  )(psum, pcnt, x_pad, wlt, wrt, b_l, w1f, b1f, w2f, b2f, w3t, b3)

  return out[:N_B1, 0]


# speedup vs baseline: 4.2251x; 1.0995x over previous
import functools

import jax
import jax.numpy as jnp
from jax import lax
from jax.experimental import pallas as pl
from jax.experimental.pallas import tpu as pltpu
from jax.experimental.pallas import tpu_sc as plsc

N_B2 = 10000
N_B1 = 10000
E = 320000
D = 128
H = 128

NC = 2
NS = 16
LANES = 16
NW = NC * NS
CHUNK = 128
CHUNKS_PER_W = 80
IBLK = 8
NBLK = CHUNKS_PER_W // IBLK
E_PAD = NW * CHUNKS_PER_W * CHUNK
N_PAD = 10240
ROWS_PER_TILE = N_PAD // NS
ROW_BLK = 512
N_BLOCKS = N_PAD // ROW_BLK


def _sc_segment_sum(x_b2, src_w, dst_w):
  mesh = plsc.VectorSubcoreMesh(core_axis_name="c", subcore_axis_name="s")

  @functools.partial(
      pl.kernel,
      out_type=(
          jax.ShapeDtypeStruct((NC, N_PAD, D), jnp.float32),
          jax.ShapeDtypeStruct((NW, N_PAD), jnp.float32),
      ),
      mesh=mesh,
      compiler_params=pltpu.CompilerParams(needs_layout_passes=False),
      scratch_types=[
          pltpu.VMEM((2, IBLK, CHUNK), jnp.int32),
          pltpu.VMEM((2, IBLK, CHUNK), jnp.int32),
          pltpu.VMEM((CHUNK, D), jnp.float32),
          pltpu.VMEM((CHUNK, D), jnp.float32),
          pltpu.VMEM((N_PAD,), jnp.float32),
          pltpu.VMEM_SHARED((N_PAD, D), jnp.float32),
          pltpu.SemaphoreType.DMA,
          pltpu.SemaphoreType.DMA,
          pltpu.SemaphoreType.DMA,
          pltpu.SemaphoreType.DMA,
          pltpu.SemaphoreType.DMA,
          pltpu.SemaphoreType.DMA,
      ],
  )
  def k(x_hbm, src_hbm, dst_hbm, psum_hbm, pcnt_hbm,
        src_b, dst_b, rows0_v, rows1_v, cnt_v, acc_sh,
        gsem0, gsem1, ssem0, ssem1, dsem0, dsem1):
    rows_v = rows0_v
    rows = (rows0_v, rows1_v)
    gsems = (gsem0, gsem1)
    ssems = (ssem0, ssem1)
    dsems = (dsem0, dsem1)
    c = lax.axis_index("c")
    s = lax.axis_index("s")
    wid = s * NC + c

    zero16 = jnp.zeros((LANES,), jnp.float32)
    one16 = jnp.ones((LANES,), jnp.float32)

    def zb(t, _):
      rows_v[t // 8, pl.ds((t % 8) * LANES, LANES)] = zero16
      return 0
    lax.fori_loop(0, CHUNK * 8, zb, 0)

    def zc(t, _):
      cnt_v[pl.ds(t * LANES, LANES)] = zero16
      return 0
    lax.fori_loop(0, N_PAD // LANES, zc, 0)

    for kk in range(ROWS_PER_TILE // CHUNK):
      pltpu.sync_copy(rows_v, acc_sh.at[pl.ds(s * ROWS_PER_TILE + kk * CHUNK, CHUNK)])
    plsc.subcore_barrier()

    def start_iblk(bb, pb):
      pltpu.async_copy(src_hbm.at[wid, pl.ds(bb * IBLK, IBLK)], src_b.at[pb],
                       ssems[pb])
      pltpu.async_copy(dst_hbm.at[wid, pl.ds(bb * IBLK, IBLK)], dst_b.at[pb],
                       dsems[pb])

    def wait_iblk(pb, which):
      if which in ("s", "both"):
        pltpu.make_async_copy(src_hbm.at[wid, pl.ds(0, IBLK)], src_b.at[pb],
                              ssems[pb]).wait()
      if which in ("d", "both"):
        pltpu.make_async_copy(dst_hbm.at[wid, pl.ds(0, IBLK)], dst_b.at[pb],
                              dsems[pb]).wait()

    start_iblk(0, 0)
    wait_iblk(0, "both")
    pltpu.async_copy(x_hbm.at[src_b.at[0, 0]], rows[0], gsems[0])

    def outer(bb2, _):
      for pb in range(2):
        bb = 2 * bb2 + pb
        np_ = 1 - pb

        @pl.when(bb + 1 < NBLK)
        def _():
          start_iblk(bb + 1, np_)

        for jj in range(IBLK):
          b = jj % 2
          nb = 1 - b
          if jj < IBLK - 1:
            pltpu.async_copy(x_hbm.at[src_b.at[pb, jj + 1]], rows[nb],
                             gsems[nb])
          else:
            @pl.when(bb + 1 < NBLK)
            def _():
              wait_iblk(np_, "both")
              pltpu.async_copy(x_hbm.at[src_b.at[np_, 0]], rows[nb],
                               gsems[nb])
          for i in range(CHUNK // LANES):
            idx = dst_b[pb, jj, pl.ds(i * LANES, LANES)]
            plsc.addupdate_scatter(cnt_v, [idx], one16)
          pltpu.make_async_copy(x_hbm.at[src_b.at[pb, jj]], rows[b],
                                gsems[b]).wait()
          pltpu.sync_copy(rows[b], acc_sh.at[dst_b.at[pb, jj]], add=True)
      return 0
    lax.fori_loop(0, NBLK // 2, outer, 0)

    plsc.subcore_barrier()

    pltpu.sync_copy(acc_sh.at[pl.ds(s * ROWS_PER_TILE, ROWS_PER_TILE)],
                    psum_hbm.at[c, pl.ds(s * ROWS_PER_TILE, ROWS_PER_TILE)])
    pltpu.sync_copy(cnt_v, pcnt_hbm.at[wid])

  return k(x_b2, src_w, dst_w)


def _tc_body(psum_ref, pcnt_ref, x_ref, wl_ref, wr_ref, bl_ref,
             w1_ref, b1_ref, w2_ref, b2_ref, w3_ref, b3_ref, out_ref):
  summed = psum_ref[0] + psum_ref[1]
  cnt = jnp.sum(pcnt_ref[...], axis=0)
  mean = summed / jnp.maximum(cnt, 1.0)[:, None]
  h = (jnp.dot(mean, wl_ref[...], preferred_element_type=jnp.float32)
       + jnp.dot(x_ref[...], wr_ref[...], preferred_element_type=jnp.float32)
       + bl_ref[...])
  h = jnp.dot(h, w1_ref[...], preferred_element_type=jnp.float32) + b1_ref[...]
  h = jnp.where(h > 0, h, 0.01 * h)
  h = jnp.dot(h, w2_ref[...], preferred_element_type=jnp.float32) + b2_ref[...]
  h = jnp.where(h > 0, h, 0.01 * h)
  out_ref[...] = jnp.dot(h, w3_ref[...], preferred_element_type=jnp.float32) + b3_ref[...]


def kernel(x_B2, x_B1, edge_index, W_l, b_l, W_r, W1, b1, g1, be1, W2, b2, g2, be2, W3, b3):
  src = jnp.concatenate([edge_index[0].astype(jnp.int32),
                         jnp.zeros((E_PAD - E,), jnp.int32)])
  pad_dst = N_B1 + (jnp.arange(E_PAD - E, dtype=jnp.int32) % (N_PAD - N_B1))
  dst = jnp.concatenate([edge_index[1].astype(jnp.int32), pad_dst])
  src_w = src.reshape(NW, CHUNKS_PER_W, CHUNK)
  dst_w = dst.reshape(NW, CHUNKS_PER_W, CHUNK)

  psum, pcnt = _sc_segment_sum(x_B2, src_w, dst_w)

  x_pad = jnp.concatenate(
      [x_B1, jnp.zeros((N_PAD - N_B1, D), jnp.float32)], axis=0)

  eps = 1e-5
  s1 = g1 / jnp.sqrt(1.0 + eps)
  s2 = g2 / jnp.sqrt(1.0 + eps)
  w1f = (W1 * s1[:, None]).T
  b1f = b1 * s1 + be1
  w2f = (W2 * s2[:, None]).T
  b2f = b2 * s2 + be2
  w3t = W3.T
  wlt = W_l.T
  wrt = W_r.T

  out = pl.pallas_call(
      _tc_body,
      grid=(N_BLOCKS,),
      in_specs=[
          pl.BlockSpec((NC, ROW_BLK, D), lambda i: (0, i, 0)),
          pl.BlockSpec((NW, ROW_BLK), lambda i: (0, i)),
          pl.BlockSpec((ROW_BLK, D), lambda i: (i, 0)),
          pl.BlockSpec((D, H), lambda i: (0, 0)),
          pl.BlockSpec((D, H), lambda i: (0, 0)),
          pl.BlockSpec((H,), lambda i: (0,)),
          pl.BlockSpec((H, 1280), lambda i: (0, 0)),
          pl.BlockSpec((1280,), lambda i: (0,)),
          pl.BlockSpec((1280, 480), lambda i: (0, 0)),
          pl.BlockSpec((480,), lambda i: (0,)),
          pl.BlockSpec((480, 1), lambda i: (0, 0)),
          pl.BlockSpec((1,), lambda i: (0,)),
      ],
      out_specs=pl.BlockSpec((ROW_BLK, 1), lambda i: (i, 0)),
      out_shape=jax.ShapeDtypeStruct((N_PAD, 1), jnp.float32),
  )(psum, pcnt, x_pad, wlt, wrt, b_l, w1f, b1f, w2f, b2f, w3t, b3)

  return out[:N_B1, 0]


# speedup vs baseline: 12.3381x; 2.9202x over previous
import functools

import jax
import jax.numpy as jnp
from jax import lax
from jax.experimental import pallas as pl
from jax.experimental.pallas import tpu as pltpu
from jax.experimental.pallas import tpu_sc as plsc

N_B2 = 10000
N_B1 = 10000
E = 320000
D = 128
H = 128

NC = 2
NS = 16
LANES = 16
NW = NC * NS
CHUNK = 128
CHUNKS_PER_W = 80
IBLK = 8
NBLK = CHUNKS_PER_W // IBLK
E_PAD = NW * CHUNKS_PER_W * CHUNK
N_PAD = 10240
ROWS_PER_TILE = N_PAD // NS
ROW_BLK = 512
N_BLOCKS = N_PAD // ROW_BLK


def _sc_segment_sum(x_b2, src_w, dst_w):
  mesh = plsc.VectorSubcoreMesh(core_axis_name="c", subcore_axis_name="s")

  @functools.partial(
      pl.kernel,
      out_type=(
          jax.ShapeDtypeStruct((NC, N_PAD, D), jnp.float32),
          jax.ShapeDtypeStruct((NW, N_PAD), jnp.float32),
      ),
      mesh=mesh,
      compiler_params=pltpu.CompilerParams(needs_layout_passes=False),
      scratch_types=[
          pltpu.VMEM((2, IBLK, CHUNK), jnp.int32),
          pltpu.VMEM((2, IBLK, CHUNK), jnp.int32),
          pltpu.VMEM((CHUNK, D), jnp.float32),
          pltpu.VMEM((CHUNK, D), jnp.float32),
          pltpu.VMEM((N_PAD,), jnp.float32),
          pltpu.VMEM_SHARED((N_PAD, D), jnp.float32),
          pltpu.SemaphoreType.DMA,
          pltpu.SemaphoreType.DMA,
          pltpu.SemaphoreType.DMA,
          pltpu.SemaphoreType.DMA,
          pltpu.SemaphoreType.DMA,
          pltpu.SemaphoreType.DMA,
      ],
  )
  def k(x_hbm, src_hbm, dst_hbm, psum_hbm, pcnt_hbm,
        src_b, dst_b, rows0_v, rows1_v, cnt_v, acc_sh,
        gsem0, gsem1, ssem0, ssem1, dsem0, dsem1):
    rows_v = rows0_v
    rows = (rows0_v, rows1_v)
    gsems = (gsem0, gsem1)
    ssems = (ssem0, ssem1)
    dsems = (dsem0, dsem1)
    c = lax.axis_index("c")
    s = lax.axis_index("s")
    wid = s * NC + c

    zero16 = jnp.zeros((LANES,), jnp.float32)
    one16 = jnp.ones((LANES,), jnp.float32)

    def zb(t, _):
      rows_v[t // 8, pl.ds((t % 8) * LANES, LANES)] = zero16
      return 0
    lax.fori_loop(0, CHUNK * 8, zb, 0)

    def zc(t, _):
      cnt_v[pl.ds(t * LANES, LANES)] = zero16
      return 0
    lax.fori_loop(0, N_PAD // LANES, zc, 0)

    for kk in range(ROWS_PER_TILE // CHUNK):
      pltpu.sync_copy(rows_v, acc_sh.at[pl.ds(s * ROWS_PER_TILE + kk * CHUNK, CHUNK)])
    plsc.subcore_barrier()

    def start_iblk(bb, pb):
      pltpu.async_copy(src_hbm.at[wid, pl.ds(bb * IBLK, IBLK)], src_b.at[pb],
                       ssems[pb])
      pltpu.async_copy(dst_hbm.at[wid, pl.ds(bb * IBLK, IBLK)], dst_b.at[pb],
                       dsems[pb])

    def wait_iblk(pb, which):
      if which in ("s", "both"):
        pltpu.make_async_copy(src_hbm.at[wid, pl.ds(0, IBLK)], src_b.at[pb],
                              ssems[pb]).wait()
      if which in ("d", "both"):
        pltpu.make_async_copy(dst_hbm.at[wid, pl.ds(0, IBLK)], dst_b.at[pb],
                              dsems[pb]).wait()

    start_iblk(0, 0)
    wait_iblk(0, "both")
    pltpu.async_copy(x_hbm.at[src_b.at[0, 0]], rows[0], gsems[0])

    def outer(bb2, _):
      for pb in range(2):
        bb = 2 * bb2 + pb
        np_ = 1 - pb

        @pl.when(bb + 1 < NBLK)
        def _():
          start_iblk(bb + 1, np_)

        for jj in range(IBLK):
          b = jj % 2
          nb = 1 - b
          if jj < IBLK - 1:
            pltpu.async_copy(x_hbm.at[src_b.at[pb, jj + 1]], rows[nb],
                             gsems[nb])
          else:
            @pl.when(bb + 1 < NBLK)
            def _():
              wait_iblk(np_, "both")
              pltpu.async_copy(x_hbm.at[src_b.at[np_, 0]], rows[nb],
                               gsems[nb])
          for i in range(CHUNK // LANES):
            idx = dst_b[pb, jj, pl.ds(i * LANES, LANES)]
            plsc.addupdate_scatter(cnt_v, [idx], one16)
          pltpu.make_async_copy(x_hbm.at[src_b.at[pb, jj]], rows[b],
                                gsems[b]).wait()
          pltpu.sync_copy(rows[b], acc_sh.at[dst_b.at[pb, jj]], add=True)
      return 0
    lax.fori_loop(0, NBLK // 2, outer, 0)

    plsc.subcore_barrier()

    pltpu.sync_copy(acc_sh.at[pl.ds(s * ROWS_PER_TILE, ROWS_PER_TILE)],
                    psum_hbm.at[c, pl.ds(s * ROWS_PER_TILE, ROWS_PER_TILE)])
    pltpu.sync_copy(cnt_v, pcnt_hbm.at[wid])

  return k(x_b2, src_w, dst_w)


def _tc_body(psum_ref, pcnt_ref, x_ref, wl_ref, wr_ref, bl_ref,
             w1_ref, b1_ref, w2_ref, b2_ref, w3_ref, b3_ref, out_ref):
  summed = psum_ref[0] + psum_ref[1]
  cnt = jnp.sum(pcnt_ref[...], axis=0)
  mean = summed / jnp.maximum(cnt, 1.0)[:, None]
  h = (jnp.dot(mean, wl_ref[...], preferred_element_type=jnp.float32)
       + jnp.dot(x_ref[...], wr_ref[...], preferred_element_type=jnp.float32)
       + bl_ref[...])
  h = jnp.dot(h, w1_ref[...], preferred_element_type=jnp.float32) + b1_ref[...]
  h = jnp.where(h > 0, h, 0.01 * h)
  h = jnp.dot(h, w2_ref[...], preferred_element_type=jnp.float32) + b2_ref[...]
  h = jnp.where(h > 0, h, 0.01 * h)
  out_ref[...] = jnp.dot(h, w3_ref[...], preferred_element_type=jnp.float32) + b3_ref[...]


def kernel(x_B2, x_B1, edge_index, W_l, b_l, W_r, W1, b1, g1, be1, W2, b2, g2, be2, W3, b3):
  pad_n = E_PAD // NW - E // NW
  pad_src = jnp.broadcast_to(jnp.arange(pad_n, dtype=jnp.int32), (NW, pad_n))
  pad_dst = pad_src + N_B1
  src = jnp.concatenate(
      [edge_index[0].astype(jnp.int32).reshape(NW, E // NW), pad_src], axis=1)
  dst = jnp.concatenate(
      [edge_index[1].astype(jnp.int32).reshape(NW, E // NW), pad_dst], axis=1)
  src_w = src.reshape(NW, CHUNKS_PER_W, CHUNK)
  dst_w = dst.reshape(NW, CHUNKS_PER_W, CHUNK)

  psum, pcnt = _sc_segment_sum(x_B2, src_w, dst_w)

  x_pad = jnp.concatenate(
      [x_B1, jnp.zeros((N_PAD - N_B1, D), jnp.float32)], axis=0)

  eps = 1e-5
  s1 = g1 / jnp.sqrt(1.0 + eps)
  s2 = g2 / jnp.sqrt(1.0 + eps)
  w1f = (W1 * s1[:, None]).T
  b1f = b1 * s1 + be1
  w2f = (W2 * s2[:, None]).T
  b2f = b2 * s2 + be2
  w3t = W3.T
  wlt = W_l.T
  wrt = W_r.T

  out = pl.pallas_call(
      _tc_body,
      grid=(N_BLOCKS,),
      in_specs=[
          pl.BlockSpec((NC, ROW_BLK, D), lambda i: (0, i, 0)),
          pl.BlockSpec((NW, ROW_BLK), lambda i: (0, i)),
          pl.BlockSpec((ROW_BLK, D), lambda i: (i, 0)),
          pl.BlockSpec((D, H), lambda i: (0, 0)),
          pl.BlockSpec((D, H), lambda i: (0, 0)),
          pl.BlockSpec((H,), lambda i: (0,)),
          pl.BlockSpec((H, 1280), lambda i: (0, 0)),
          pl.BlockSpec((1280,), lambda i: (0,)),
          pl.BlockSpec((1280, 480), lambda i: (0, 0)),
          pl.BlockSpec((480,), lambda i: (0,)),
          pl.BlockSpec((480, 1), lambda i: (0, 0)),
          pl.BlockSpec((1,), lambda i: (0,)),
      ],
      out_specs=pl.BlockSpec((ROW_BLK, 1), lambda i: (i, 0)),
      out_shape=jax.ShapeDtypeStruct((N_PAD, 1), jnp.float32),
  )(psum, pcnt, x_pad, wlt, wrt, b_l, w1f, b1f, w2f, b2f, w3t, b3)

  return out[:N_B1, 0]
